# Initial kernel scaffold; baseline (speedup 1.0000x reference)
import functools

import jax
import jax.numpy as jnp
from jax import lax
from jax.experimental import pallas as pl
from jax.experimental.pallas import tpu as pltpu
from jax.experimental.pallas import tpu_sc as plsc

_NC = 2
_NS = 16
_L = 16
_FPT = 4
_CH = 2000

_F32 = jnp.float32


def _elu(x):
    return jnp.where(x > 0, x, jnp.expm1(x))


def _entry_body(xT_ref, w_ref, as_ref, ad_ref, zT_ref, es_ref, ed_ref, g_ref):
    xT = xT_ref[...]
    w = w_ref[...]
    zT = lax.dot_general(w, xT, (((0,), (0,)), ((), ())),
                         preferred_element_type=_F32)
    zT_ref[...] = zT
    es = jnp.dot(as_ref[...], zT, preferred_element_type=_F32)
    ed = jnp.dot(ad_ref[...], zT, preferred_element_type=_F32)
    es_ref[...] = es
    ed_ref[...] = ed
    g_ref[...] = jnp.full((1, _L), jnp.max(es), _F32)


def _tc_entry(xT, W0, a_s, a_d):
    D, N = xT.shape
    H = W0.shape[1]
    return pl.pallas_call(
        _entry_body,
        out_shape=(
            jax.ShapeDtypeStruct((H, N), _F32),
            jax.ShapeDtypeStruct((1, N), _F32),
            jax.ShapeDtypeStruct((1, N), _F32),
            jax.ShapeDtypeStruct((1, _L), _F32),
        ),
    )(xT, W0, a_s.reshape(1, -1), a_d.reshape(1, -1))


def _mid_body(transpose_w, layer4, emit_h, refs):
    if emit_h:
        (num_ref, sp_ref, w_ref, as_ref, ad_ref,
         zT_ref, es_ref, ed_ref, g_ref, hT_ref) = refs
    else:
        (num_ref, sp_ref, w_ref, as_ref, ad_ref,
         zT_ref, es_ref, ed_ref, g_ref) = refs
    s = jnp.sum(sp_ref[...], axis=0, keepdims=True)
    num = jnp.sum(num_ref[...], axis=0)
    hT = _elu(num / (s + 1e-9))
    if emit_h:
        hT_ref[...] = hT
    if layer4:
        zT_ref[...] = hT
        wa_s = jnp.dot(as_ref[...], w_ref[...], preferred_element_type=_F32)
        wa_d = jnp.dot(ad_ref[...], w_ref[...], preferred_element_type=_F32)
        es = jnp.dot(wa_s, hT, preferred_element_type=_F32)
        ed = jnp.dot(wa_d, hT, preferred_element_type=_F32)
    else:
        cdim = 1 if transpose_w else 0
        zT = lax.dot_general(w_ref[...], hT, (((cdim,), (0,)), ((), ())),
                             preferred_element_type=_F32)
        zT_ref[...] = zT
        es = jnp.dot(as_ref[...], zT, preferred_element_type=_F32)
        ed = jnp.dot(ad_ref[...], zT, preferred_element_type=_F32)
    es_ref[...] = es
    ed_ref[...] = ed
    g_ref[...] = jnp.full((1, _L), jnp.max(es), _F32)


def _tc_mid(num, sp, W, a_s, a_d, *, transpose_w, layer4, emit_h):
    N = sp.shape[1]
    H = 16
    out_shape = [
        jax.ShapeDtypeStruct((H, N), _F32),
        jax.ShapeDtypeStruct((1, N), _F32),
        jax.ShapeDtypeStruct((1, N), _F32),
        jax.ShapeDtypeStruct((1, _L), _F32),
    ]
    if emit_h:
        out_shape.append(jax.ShapeDtypeStruct((16, N), _F32))

    def body(*refs):
        _mid_body(transpose_w, layer4, emit_h, refs)

    return pl.pallas_call(body, out_shape=tuple(out_shape))(
        num, sp, W, a_s.reshape(1, -1), a_d.reshape(1, -1))


def _final_body(num_ref, sp_ref, w_ref, out_ref):
    s = jnp.sum(sp_ref[...], axis=0, keepdims=True)
    agg = jnp.sum(num_ref[...], axis=0) / (s + 1e-9)
    reconT = lax.dot_general(w_ref[...], agg, (((1,), (0,)), ((), ())),
                             preferred_element_type=_F32)
    out_ref[...] = _elu(reconT)


def _tc_final(num, sp, W0):
    N = sp.shape[1]
    D = W0.shape[0]
    return pl.pallas_call(
        _final_body,
        out_shape=jax.ShapeDtypeStruct((D, N), _F32),
    )(num, sp, W0)


def _t_body(x_ref, o_ref):
    o_ref[...] = x_ref[...].T


def _transpose_big(x):
    A, B = x.shape
    bn = 2000
    if A >= B:
        assert A % bn == 0
        return pl.pallas_call(
            _t_body,
            grid=(A // bn,),
            in_specs=[pl.BlockSpec((bn, B), lambda i: (i, 0))],
            out_specs=pl.BlockSpec((B, bn), lambda i: (0, i)),
            out_shape=jax.ShapeDtypeStruct((B, A), x.dtype),
        )(x)
    assert B % bn == 0
    return pl.pallas_call(
        _t_body,
        grid=(B // bn,),
        in_specs=[pl.BlockSpec((A, bn), lambda i: (0, i))],
        out_specs=pl.BlockSpec((bn, A), lambda i: (i, 0)),
        out_shape=jax.ShapeDtypeStruct((B, A), x.dtype),
    )(x)


def _sc_edge_body(N, E, src_hbm, dst_hbm, es_hbm, ed_hbm, g_hbm, vT_hbm,
                  num_hbm, s_hbm,
                  es_v, ed_v, g_v, z_v, acc_v, s_v, src_v, dst_v, ex_v, ex_sh):
    e_half = E // _NC
    ept1 = e_half // _NS
    eq_sz = e_half // _FPT
    ngrp = _CH // _L

    c = lax.axis_index("c")
    t = lax.axis_index("s")
    e0 = c * e_half

    pltpu.sync_copy(es_hbm, es_v)
    pltpu.sync_copy(ed_hbm, ed_v)
    pltpu.sync_copy(g_hbm, g_v)

    def zero_s(i, _):
        s_v[pl.ds(i * _L, _L)] = jnp.zeros((_L,), _F32)
        return 0

    lax.fori_loop(0, N // _L, zero_s, 0)
    g = g_v[...]

    p1base = e0 + t * ept1

    def p1_chunk(k, _):
        b = p1base + k * _CH
        pltpu.sync_copy(src_hbm.at[pl.ds(b, _CH)], src_v)
        pltpu.sync_copy(dst_hbm.at[pl.ds(b, _CH)], dst_v)

        def grp(gi, _):
            sv = src_v[pl.ds(gi * _L, _L)]
            dv = dst_v[pl.ds(gi * _L, _L)]
            a = plsc.load_gather(es_v, [sv])
            bd = plsc.load_gather(ed_v, [dv])
            u = a + bd
            e = jnp.maximum(u, 0.2 * u)
            tq = bd + g
            q = jnp.maximum(tq, 0.2 * tq)
            ex = jnp.exp(e - q)
            ex_v[pl.ds(gi * _L, _L)] = ex
            plsc.addupdate_scatter(s_v, [dv], ex)
            return 0

        lax.fori_loop(0, ngrp, grp, 0)
        pltpu.sync_copy(ex_v, ex_sh.at[pl.ds(t * ept1 + k * _CH, _CH)])
        return 0

    lax.fori_loop(0, ept1 // _CH, p1_chunk, 0)
    pltpu.sync_copy(s_v, s_hbm.at[c * _NS + t])
    plsc.subcore_barrier()

    fq = t % (_NS // _FPT)
    eq = t // (_NS // _FPT)
    fb = fq * _FPT
    pltpu.sync_copy(vT_hbm.at[pl.ds(fb, _FPT)], z_v)

    def zero_acc(i, _):
        for j in range(_FPT):
            acc_v[j, pl.ds(i * _L, _L)] = jnp.zeros((_L,), _F32)
        return 0

    lax.fori_loop(0, N // _L, zero_acc, 0)

    p2local = eq * eq_sz

    def p2_chunk(k, _):
        bl = p2local + k * _CH
        b = e0 + bl
        pltpu.sync_copy(src_hbm.at[pl.ds(b, _CH)], src_v)
        pltpu.sync_copy(dst_hbm.at[pl.ds(b, _CH)], dst_v)
        pltpu.sync_copy(ex_sh.at[pl.ds(bl, _CH)], ex_v)

        def grp(gi, _):
            sv = src_v[pl.ds(gi * _L, _L)]
            dv = dst_v[pl.ds(gi * _L, _L)]
            exv = ex_v[pl.ds(gi * _L, _L)]
            for j in range(_FPT):
                rj = jnp.full((_L,), j, jnp.int32)
                zg = plsc.load_gather(z_v, [rj, sv])
                plsc.addupdate_scatter(acc_v, [rj, dv], zg * exv)
            return 0

        lax.fori_loop(0, ngrp, grp, 0)
        return 0

    lax.fori_loop(0, eq_sz // _CH, p2_chunk, 0)
    pltpu.sync_copy(acc_v, num_hbm.at[c * _FPT + eq, pl.ds(fb, _FPT)])


@functools.partial(jax.jit, static_argnames=("N", "E"))
def _sc_edge(src, dst, es, ed, g, vT, *, N, E):
    mesh = plsc.VectorSubcoreMesh(core_axis_name="c", subcore_axis_name="s",
                                  num_cores=_NC, num_subcores=_NS)
    body = functools.partial(_sc_edge_body, N, E)
    kern = pl.kernel(
        body,
        out_type=(
            jax.ShapeDtypeStruct((_NC * _FPT, 16, N), _F32),
            jax.ShapeDtypeStruct((_NC * _NS, N), _F32),
        ),
        mesh=mesh,
        scratch_types=[
            pltpu.VMEM((N,), _F32),
            pltpu.VMEM((N,), _F32),
            pltpu.VMEM((_L,), _F32),
            pltpu.VMEM((_FPT, N), _F32),
            pltpu.VMEM((_FPT, N), _F32),
            pltpu.VMEM((N,), _F32),
            pltpu.VMEM((_CH,), jnp.int32),
            pltpu.VMEM((_CH,), jnp.int32),
            pltpu.VMEM((_CH,), _F32),
            pltpu.VMEM_SHARED((E // _NC,), _F32),
        ],
    )
    return kern(src, dst, es, ed, g, vT)


def kernel(x, edge_index, W0, W1,
           a_enc0_src, a_enc0_dst, a_enc1_src, a_enc1_dst,
           a_dec0_src, a_dec0_dst, a_dec1_src, a_dec1_dst):
    N, D = x.shape
    E = edge_index.shape[1]
    src = edge_index[0]
    dst = edge_index[1]

    xT = _transpose_big(x)

    zT, es, ed, g = _tc_entry(xT, W0, a_enc0_src, a_enc0_dst)
    num, sp = _sc_edge(src, dst, es.reshape(-1), ed.reshape(-1),
                       g.reshape(-1), zT, N=N, E=E)

    zT, es, ed, g = _tc_mid(num, sp, W1, a_enc1_src, a_enc1_dst,
                            transpose_w=False, layer4=False, emit_h=False)
    num, sp = _sc_edge(src, dst, es.reshape(-1), ed.reshape(-1),
                       g.reshape(-1), zT, N=N, E=E)

    zT, es, ed, g, hiddenT = _tc_mid(num, sp, W1, a_dec0_src, a_dec0_dst,
                                     transpose_w=True, layer4=False,
                                     emit_h=True)
    num, sp = _sc_edge(src, dst, es.reshape(-1), ed.reshape(-1),
                       g.reshape(-1), zT, N=N, E=E)

    rT, es, ed, g = _tc_mid(num, sp, W0, a_dec1_src, a_dec1_dst,
                            transpose_w=False, layer4=True, emit_h=False)
    num, sp = _sc_edge(src, dst, es.reshape(-1), ed.reshape(-1),
                       g.reshape(-1), rT, N=N, E=E)
    reconT = _tc_final(num, sp, W0)

    hidden = _transpose_big(hiddenT)
    recon = _transpose_big(reconT)
    return (hidden, recon)


# speedup vs baseline: 46.1342x; 46.1342x over previous
import functools

import jax
import jax.numpy as jnp
from jax import lax
from jax.experimental import pallas as pl
from jax.experimental.pallas import tpu as pltpu
from jax.experimental.pallas import tpu_sc as plsc

_NC = 2
_NS = 16
_L = 16
_FPT = 4
_CH = 2000

_F32 = jnp.float32


def _elu(x):
    return jnp.where(x > 0, x, jnp.exp(jnp.minimum(x, 0.0)) - 1.0)


def _entry_body(xT_ref, w_ref, as_ref, ad_ref, zT_ref, es_ref, ed_ref, g_ref):
    xT = xT_ref[...]
    w = w_ref[...]
    zT = lax.dot_general(w, xT, (((0,), (0,)), ((), ())),
                         preferred_element_type=_F32)
    zT_ref[...] = zT
    es = jnp.dot(as_ref[...], zT, preferred_element_type=_F32)
    ed = jnp.dot(ad_ref[...], zT, preferred_element_type=_F32)
    es_ref[...] = es
    ed_ref[...] = ed
    g_ref[...] = jnp.full((1, _L), jnp.max(es), _F32)


def _tc_entry(xT, W0, a_s, a_d):
    D, N = xT.shape
    H = W0.shape[1]
    return pl.pallas_call(
        _entry_body,
        out_shape=(
            jax.ShapeDtypeStruct((H, N), _F32),
            jax.ShapeDtypeStruct((1, N), _F32),
            jax.ShapeDtypeStruct((1, N), _F32),
            jax.ShapeDtypeStruct((1, _L), _F32),
        ),
    )(xT, W0, a_s.reshape(1, -1), a_d.reshape(1, -1))


def _mid_body(transpose_w, layer4, emit_h, refs):
    if emit_h:
        (num_ref, sp_ref, w_ref, as_ref, ad_ref,
         zT_ref, es_ref, ed_ref, g_ref, hT_ref) = refs
    else:
        (num_ref, sp_ref, w_ref, as_ref, ad_ref,
         zT_ref, es_ref, ed_ref, g_ref) = refs
    s = jnp.sum(sp_ref[...], axis=0, keepdims=True)
    num = jnp.sum(num_ref[...], axis=0)
    hT = _elu(num / (s + 1e-9))
    if emit_h:
        hT_ref[...] = hT
    if layer4:
        zT_ref[...] = hT
        wa_s = jnp.dot(as_ref[...], w_ref[...], preferred_element_type=_F32)
        wa_d = jnp.dot(ad_ref[...], w_ref[...], preferred_element_type=_F32)
        es = jnp.dot(wa_s, hT, preferred_element_type=_F32)
        ed = jnp.dot(wa_d, hT, preferred_element_type=_F32)
    else:
        cdim = 1 if transpose_w else 0
        zT = lax.dot_general(w_ref[...], hT, (((cdim,), (0,)), ((), ())),
                             preferred_element_type=_F32)
        zT_ref[...] = zT
        es = jnp.dot(as_ref[...], zT, preferred_element_type=_F32)
        ed = jnp.dot(ad_ref[...], zT, preferred_element_type=_F32)
    es_ref[...] = es
    ed_ref[...] = ed
    g_ref[...] = jnp.full((1, _L), jnp.max(es), _F32)


def _tc_mid(num, sp, W, a_s, a_d, *, transpose_w, layer4, emit_h):
    N = sp.shape[1]
    H = 16
    out_shape = [
        jax.ShapeDtypeStruct((H, N), _F32),
        jax.ShapeDtypeStruct((1, N), _F32),
        jax.ShapeDtypeStruct((1, N), _F32),
        jax.ShapeDtypeStruct((1, _L), _F32),
    ]
    if emit_h:
        out_shape.append(jax.ShapeDtypeStruct((16, N), _F32))

    def body(*refs):
        _mid_body(transpose_w, layer4, emit_h, refs)

    return pl.pallas_call(body, out_shape=tuple(out_shape))(
        num, sp, W, a_s.reshape(1, -1), a_d.reshape(1, -1))


def _final_body(num_ref, sp_ref, w_ref, out_ref):
    s = jnp.sum(sp_ref[...], axis=0, keepdims=True)
    agg = jnp.sum(num_ref[...], axis=0) / (s + 1e-9)
    reconT = lax.dot_general(w_ref[...], agg, (((1,), (0,)), ((), ())),
                             preferred_element_type=_F32)
    out_ref[...] = _elu(reconT)


def _tc_final(num, sp, W0):
    N = sp.shape[1]
    D = W0.shape[0]
    return pl.pallas_call(
        _final_body,
        out_shape=jax.ShapeDtypeStruct((D, N), _F32),
    )(num, sp, W0)


def _t_body(x_ref, o_ref):
    o_ref[...] = x_ref[...].T


def _transpose_big(x):
    A, B = x.shape
    return pl.pallas_call(
        _t_body,
        out_shape=jax.ShapeDtypeStruct((B, A), x.dtype),
    )(x)


def _sc_edge_body(N, E, src_hbm, dst_hbm, es_hbm, ed_hbm, g_hbm, vT_hbm,
                  num_hbm, s_hbm,
                  es_v, ed_v, g_v, z_v, acc_v, s_v, src_v, dst_v, ex_v, ex_sh):
    e_half = E // _NC
    ept1 = e_half // _NS
    eq_sz = e_half // _FPT
    ngrp = _CH // _L

    c = lax.axis_index("c")
    t = lax.axis_index("s")
    e0 = c * e_half

    pltpu.sync_copy(es_hbm, es_v)
    pltpu.sync_copy(ed_hbm, ed_v)
    pltpu.sync_copy(g_hbm, g_v)

    def zero_s(i, _):
        s_v[pl.ds(i * _L, _L)] = jnp.zeros((_L,), _F32)
        return 0

    lax.fori_loop(0, N // _L, zero_s, 0)
    g = g_v[...]

    p1base = e0 + t * ept1

    def p1_chunk(k, _):
        b = p1base + k * _CH
        pltpu.sync_copy(src_hbm.at[pl.ds(b, _CH)], src_v)
        pltpu.sync_copy(dst_hbm.at[pl.ds(b, _CH)], dst_v)

        def grp(gi, _):
            sv = src_v[pl.ds(gi * _L, _L)]
            dv = dst_v[pl.ds(gi * _L, _L)]
            a = plsc.load_gather(es_v, [sv])
            bd = plsc.load_gather(ed_v, [dv])
            u = a + bd
            e = jnp.maximum(u, 0.2 * u)
            tq = bd + g
            q = jnp.maximum(tq, 0.2 * tq)
            ex = jnp.exp(e - q)
            ex_v[pl.ds(gi * _L, _L)] = ex
            plsc.addupdate_scatter(s_v, [dv], ex)
            return 0

        lax.fori_loop(0, ngrp, grp, 0)
        pltpu.sync_copy(ex_v, ex_sh.at[pl.ds(t * ept1 + k * _CH, _CH)])
        return 0

    lax.fori_loop(0, ept1 // _CH, p1_chunk, 0)
    pltpu.sync_copy(s_v, s_hbm.at[c * _NS + t])
    plsc.subcore_barrier()

    fq = t % (_NS // _FPT)
    eq = t // (_NS // _FPT)
    fb = fq * _FPT
    pltpu.sync_copy(vT_hbm.at[pl.ds(fb, _FPT)], z_v)

    def zero_acc(i, _):
        for j in range(_FPT):
            acc_v[j, pl.ds(i * _L, _L)] = jnp.zeros((_L,), _F32)
        return 0

    lax.fori_loop(0, N // _L, zero_acc, 0)

    p2local = eq * eq_sz

    def p2_chunk(k, _):
        bl = p2local + k * _CH
        b = e0 + bl
        pltpu.sync_copy(src_hbm.at[pl.ds(b, _CH)], src_v)
        pltpu.sync_copy(dst_hbm.at[pl.ds(b, _CH)], dst_v)
        pltpu.sync_copy(ex_sh.at[pl.ds(bl, _CH)], ex_v)

        def grp(gi, _):
            sv = src_v[pl.ds(gi * _L, _L)]
            dv = dst_v[pl.ds(gi * _L, _L)]
            exv = ex_v[pl.ds(gi * _L, _L)]
            for j in range(_FPT):
                rj = jnp.full((_L,), j, jnp.int32)
                zg = plsc.load_gather(z_v, [rj, sv])
                plsc.addupdate_scatter(acc_v, [rj, dv], zg * exv)
            return 0

        lax.fori_loop(0, ngrp, grp, 0)
        return 0

    lax.fori_loop(0, eq_sz // _CH, p2_chunk, 0)
    pltpu.sync_copy(acc_v, num_hbm.at[c * _FPT + eq, pl.ds(fb, _FPT)])


@functools.partial(jax.jit, static_argnames=("N", "E"))
def _sc_edge(src, dst, es, ed, g, vT, *, N, E):
    mesh = plsc.VectorSubcoreMesh(core_axis_name="c", subcore_axis_name="s",
                                  num_cores=_NC, num_subcores=_NS)
    body = functools.partial(_sc_edge_body, N, E)
    kern = pl.kernel(
        body,
        out_type=(
            jax.ShapeDtypeStruct((_NC * _FPT, 16, N), _F32),
            jax.ShapeDtypeStruct((_NC * _NS, N), _F32),
        ),
        mesh=mesh,
        compiler_params=pltpu.CompilerParams(needs_layout_passes=False),
        scratch_types=[
            pltpu.VMEM((N,), _F32),
            pltpu.VMEM((N,), _F32),
            pltpu.VMEM((_L,), _F32),
            pltpu.VMEM((_FPT, N), _F32),
            pltpu.VMEM((_FPT, N), _F32),
            pltpu.VMEM((N,), _F32),
            pltpu.VMEM((_CH,), jnp.int32),
            pltpu.VMEM((_CH,), jnp.int32),
            pltpu.VMEM((_CH,), _F32),
            pltpu.VMEM_SHARED((E // _NC,), _F32),
        ],
    )
    return kern(src, dst, es, ed, g, vT)


def kernel(x, edge_index, W0, W1,
           a_enc0_src, a_enc0_dst, a_enc1_src, a_enc1_dst,
           a_dec0_src, a_dec0_dst, a_dec1_src, a_dec1_dst):
    N, D = x.shape
    E = edge_index.shape[1]
    src = edge_index[0]
    dst = edge_index[1]

    xT = _transpose_big(x)

    zT, es, ed, g = _tc_entry(xT, W0, a_enc0_src, a_enc0_dst)
    num, sp = _sc_edge(src, dst, es.reshape(-1), ed.reshape(-1),
                       g.reshape(-1), zT, N=N, E=E)

    zT, es, ed, g = _tc_mid(num, sp, W1, a_enc1_src, a_enc1_dst,
                            transpose_w=False, layer4=False, emit_h=False)
    num, sp = _sc_edge(src, dst, es.reshape(-1), ed.reshape(-1),
                       g.reshape(-1), zT, N=N, E=E)

    zT, es, ed, g, hiddenT = _tc_mid(num, sp, W1, a_dec0_src, a_dec0_dst,
                                     transpose_w=True, layer4=False,
                                     emit_h=True)
    num, sp = _sc_edge(src, dst, es.reshape(-1), ed.reshape(-1),
                       g.reshape(-1), zT, N=N, E=E)

    rT, es, ed, g = _tc_mid(num, sp, W0, a_dec1_src, a_dec1_dst,
                            transpose_w=False, layer4=True, emit_h=False)
    num, sp = _sc_edge(src, dst, es.reshape(-1), ed.reshape(-1),
                       g.reshape(-1), rT, N=N, E=E)
    reconT = _tc_final(num, sp, W0)

    hidden = _transpose_big(hiddenT)
    recon = _transpose_big(reconT)
    return (hidden, recon)


# speedup vs baseline: 46.4878x; 1.0077x over previous
import functools

import jax
import jax.numpy as jnp
from jax import lax
from jax.experimental import pallas as pl
from jax.experimental.pallas import tpu as pltpu
from jax.experimental.pallas import tpu_sc as plsc

_NC = 2
_NS = 16
_L = 16
_FPT = 4
_CH = 2000

_F32 = jnp.float32


def _elu(x):
    return jnp.where(x > 0, x, jnp.exp(jnp.minimum(x, 0.0)) - 1.0)


def _entry_body(xT_ref, w_ref, as_ref, ad_ref, zT_ref, es_ref, ed_ref, g_ref):
    xT = xT_ref[...]
    w = w_ref[...]
    zT = lax.dot_general(w, xT, (((0,), (0,)), ((), ())),
                         preferred_element_type=_F32)
    zT_ref[...] = zT
    es = jnp.dot(as_ref[...], zT, preferred_element_type=_F32)
    ed = jnp.dot(ad_ref[...], zT, preferred_element_type=_F32)
    es_ref[...] = es
    ed_ref[...] = ed
    g_ref[...] = jnp.full((1, _L), jnp.max(es), _F32)


def _tc_entry(xT, W0, a_s, a_d):
    D, N = xT.shape
    H = W0.shape[1]
    return pl.pallas_call(
        _entry_body,
        out_shape=(
            jax.ShapeDtypeStruct((H, N), _F32),
            jax.ShapeDtypeStruct((1, N), _F32),
            jax.ShapeDtypeStruct((1, N), _F32),
            jax.ShapeDtypeStruct((1, _L), _F32),
        ),
    )(xT, W0, a_s.reshape(1, -1), a_d.reshape(1, -1))


def _mid_body(transpose_w, layer4, emit_h, refs):
    if emit_h:
        (num_ref, sp_ref, w_ref, as_ref, ad_ref,
         zT_ref, es_ref, ed_ref, g_ref, hT_ref) = refs
    else:
        (num_ref, sp_ref, w_ref, as_ref, ad_ref,
         zT_ref, es_ref, ed_ref, g_ref) = refs
    s = jnp.sum(sp_ref[...], axis=0, keepdims=True)
    num = jnp.sum(num_ref[...], axis=0)
    hT = _elu(num / (s + 1e-9))
    if emit_h:
        hT_ref[...] = hT
    if layer4:
        zT_ref[...] = hT
        wa_s = jnp.dot(as_ref[...], w_ref[...], preferred_element_type=_F32)
        wa_d = jnp.dot(ad_ref[...], w_ref[...], preferred_element_type=_F32)
        es = jnp.dot(wa_s, hT, preferred_element_type=_F32)
        ed = jnp.dot(wa_d, hT, preferred_element_type=_F32)
    else:
        cdim = 1 if transpose_w else 0
        zT = lax.dot_general(w_ref[...], hT, (((cdim,), (0,)), ((), ())),
                             preferred_element_type=_F32)
        zT_ref[...] = zT
        es = jnp.dot(as_ref[...], zT, preferred_element_type=_F32)
        ed = jnp.dot(ad_ref[...], zT, preferred_element_type=_F32)
    es_ref[...] = es
    ed_ref[...] = ed
    g_ref[...] = jnp.full((1, _L), jnp.max(es), _F32)


def _tc_mid(num, sp, W, a_s, a_d, *, transpose_w, layer4, emit_h):
    N = sp.shape[1]
    H = 16
    out_shape = [
        jax.ShapeDtypeStruct((H, N), _F32),
        jax.ShapeDtypeStruct((1, N), _F32),
        jax.ShapeDtypeStruct((1, N), _F32),
        jax.ShapeDtypeStruct((1, _L), _F32),
    ]
    if emit_h:
        out_shape.append(jax.ShapeDtypeStruct((16, N), _F32))

    def body(*refs):
        _mid_body(transpose_w, layer4, emit_h, refs)

    return pl.pallas_call(body, out_shape=tuple(out_shape))(
        num, sp, W, a_s.reshape(1, -1), a_d.reshape(1, -1))


def _final_body(num_ref, sp_ref, w_ref, out_ref):
    s = jnp.sum(sp_ref[...], axis=0, keepdims=True)
    agg = jnp.sum(num_ref[...], axis=0) / (s + 1e-9)
    reconT = lax.dot_general(w_ref[...], agg, (((1,), (0,)), ((), ())),
                             preferred_element_type=_F32)
    out_ref[...] = _elu(reconT)


def _tc_final(num, sp, W0):
    N = sp.shape[1]
    D = W0.shape[0]
    return pl.pallas_call(
        _final_body,
        out_shape=jax.ShapeDtypeStruct((D, N), _F32),
    )(num, sp, W0)


def _t_body(x_ref, o_ref):
    o_ref[...] = x_ref[...].T


def _transpose_big(x):
    A, B = x.shape
    return pl.pallas_call(
        _t_body,
        out_shape=jax.ShapeDtypeStruct((B, A), x.dtype),
    )(x)


def _sc_edge_body(N, E, src_hbm, dst_hbm, es_hbm, ed_hbm, g_hbm, vT_hbm,
                  num_hbm, s_hbm,
                  es_v, ed_v, g_v, z_v, acc_v, s_v, src_v, dst_v, ex_v, ex_sh):
    e_half = E // _NC
    ept1 = e_half // _NS
    eq_sz = e_half // _FPT
    ngrp = _CH // _L

    c = lax.axis_index("c")
    t = lax.axis_index("s")
    e0 = c * e_half

    pltpu.sync_copy(es_hbm, es_v)
    pltpu.sync_copy(ed_hbm, ed_v)
    pltpu.sync_copy(g_hbm, g_v)

    def zero_s(i, _):
        for u in range(5):
            s_v[pl.ds((i * 5 + u) * _L, _L)] = jnp.zeros((_L,), _F32)
        return 0

    lax.fori_loop(0, N // _L // 5, zero_s, 0)
    g = g_v[...]

    p1base = e0 + t * ept1

    def p1_chunk(k, _):
        b = p1base + k * _CH
        pltpu.sync_copy(src_hbm.at[pl.ds(b, _CH)], src_v)
        pltpu.sync_copy(dst_hbm.at[pl.ds(b, _CH)], dst_v)

        def grp(gi, _):
            for uu in range(5):
                o = (gi * 5 + uu) * _L
                sv = src_v[pl.ds(o, _L)]
                dv = dst_v[pl.ds(o, _L)]
                a = plsc.load_gather(es_v, [sv])
                bd = plsc.load_gather(ed_v, [dv])
                u = a + bd
                e = jnp.maximum(u, 0.2 * u)
                tq = bd + g
                q = jnp.maximum(tq, 0.2 * tq)
                ex = jnp.exp(e - q)
                ex_v[pl.ds(o, _L)] = ex
                plsc.addupdate_scatter(s_v, [dv], ex)
            return 0

        lax.fori_loop(0, ngrp // 5, grp, 0)
        pltpu.sync_copy(ex_v, ex_sh.at[pl.ds(t * ept1 + k * _CH, _CH)])
        return 0

    lax.fori_loop(0, ept1 // _CH, p1_chunk, 0)
    pltpu.sync_copy(s_v, s_hbm.at[c * _NS + t])
    plsc.subcore_barrier()

    fq = t % (_NS // _FPT)
    eq = t // (_NS // _FPT)
    fb = fq * _FPT
    pltpu.sync_copy(vT_hbm.at[pl.ds(fb, _FPT)], z_v)

    def zero_acc(i, _):
        for u in range(5):
            for j in range(_FPT):
                acc_v[j, pl.ds((i * 5 + u) * _L, _L)] = jnp.zeros((_L,), _F32)
        return 0

    lax.fori_loop(0, N // _L // 5, zero_acc, 0)

    p2local = eq * eq_sz

    def p2_chunk(k, _):
        bl = p2local + k * _CH
        b = e0 + bl
        pltpu.sync_copy(src_hbm.at[pl.ds(b, _CH)], src_v)
        pltpu.sync_copy(dst_hbm.at[pl.ds(b, _CH)], dst_v)
        pltpu.sync_copy(ex_sh.at[pl.ds(bl, _CH)], ex_v)

        def grp(gi, _):
            for uu in range(5):
                o = (gi * 5 + uu) * _L
                sv = src_v[pl.ds(o, _L)]
                dv = dst_v[pl.ds(o, _L)]
                exv = ex_v[pl.ds(o, _L)]
                for j in range(_FPT):
                    rj = jnp.full((_L,), j, jnp.int32)
                    zg = plsc.load_gather(z_v, [rj, sv])
                    plsc.addupdate_scatter(acc_v, [rj, dv], zg * exv)
            return 0

        lax.fori_loop(0, ngrp // 5, grp, 0)
        return 0

    lax.fori_loop(0, eq_sz // _CH, p2_chunk, 0)
    pltpu.sync_copy(acc_v, num_hbm.at[c * _FPT + eq, pl.ds(fb, _FPT)])


@functools.partial(jax.jit, static_argnames=("N", "E"))
def _sc_edge(src, dst, es, ed, g, vT, *, N, E):
    mesh = plsc.VectorSubcoreMesh(core_axis_name="c", subcore_axis_name="s",
                                  num_cores=_NC, num_subcores=_NS)
    body = functools.partial(_sc_edge_body, N, E)
    kern = pl.kernel(
        body,
        out_type=(
            jax.ShapeDtypeStruct((_NC * _FPT, 16, N), _F32),
            jax.ShapeDtypeStruct((_NC * _NS, N), _F32),
        ),
        mesh=mesh,
        compiler_params=pltpu.CompilerParams(needs_layout_passes=False),
        scratch_types=[
            pltpu.VMEM((N,), _F32),
            pltpu.VMEM((N,), _F32),
            pltpu.VMEM((_L,), _F32),
            pltpu.VMEM((_FPT, N), _F32),
            pltpu.VMEM((_FPT, N), _F32),
            pltpu.VMEM((N,), _F32),
            pltpu.VMEM((_CH,), jnp.int32),
            pltpu.VMEM((_CH,), jnp.int32),
            pltpu.VMEM((_CH,), _F32),
            pltpu.VMEM_SHARED((E // _NC,), _F32),
        ],
    )
    return kern(src, dst, es, ed, g, vT)


def kernel(x, edge_index, W0, W1,
           a_enc0_src, a_enc0_dst, a_enc1_src, a_enc1_dst,
           a_dec0_src, a_dec0_dst, a_dec1_src, a_dec1_dst):
    N, D = x.shape
    E = edge_index.shape[1]
    src = edge_index[0]
    dst = edge_index[1]

    xT = _transpose_big(x)

    zT, es, ed, g = _tc_entry(xT, W0, a_enc0_src, a_enc0_dst)
    num, sp = _sc_edge(src, dst, es.reshape(-1), ed.reshape(-1),
                       g.reshape(-1), zT, N=N, E=E)

    zT, es, ed, g = _tc_mid(num, sp, W1, a_enc1_src, a_enc1_dst,
                            transpose_w=False, layer4=False, emit_h=False)
    num, sp = _sc_edge(src, dst, es.reshape(-1), ed.reshape(-1),
                       g.reshape(-1), zT, N=N, E=E)

    zT, es, ed, g, hiddenT = _tc_mid(num, sp, W1, a_dec0_src, a_dec0_dst,
                                     transpose_w=True, layer4=False,
                                     emit_h=True)
    num, sp = _sc_edge(src, dst, es.reshape(-1), ed.reshape(-1),
                       g.reshape(-1), zT, N=N, E=E)

    rT, es, ed, g = _tc_mid(num, sp, W0, a_dec1_src, a_dec1_dst,
                            transpose_w=False, layer4=True, emit_h=False)
    num, sp = _sc_edge(src, dst, es.reshape(-1), ed.reshape(-1),
                       g.reshape(-1), rT, N=N, E=E)
    reconT = _tc_final(num, sp, W0)

    hidden = _transpose_big(hiddenT)
    recon = _transpose_big(reconT)
    return (hidden, recon)


# speedup vs baseline: 76.9465x; 1.6552x over previous
import functools

import jax
import jax.numpy as jnp
from jax import lax
from jax.experimental import pallas as pl
from jax.experimental.pallas import tpu as pltpu
from jax.experimental.pallas import tpu_sc as plsc

_NC = 2
_NS = 16
_L = 16
_FPT = 4
_CH = 2000

_F32 = jnp.float32


def _elu(x):
    return jnp.where(x > 0, x, jnp.exp(jnp.minimum(x, 0.0)) - 1.0)


def _entry_body(xT_ref, w_ref, as_ref, ad_ref, zT_ref, es_ref, ed_ref, g_ref):
    xT = xT_ref[...]
    w = w_ref[...]
    zT = lax.dot_general(w, xT, (((0,), (0,)), ((), ())),
                         preferred_element_type=_F32)
    zT_ref[...] = zT
    es = jnp.dot(as_ref[...], zT, preferred_element_type=_F32)
    ed = jnp.dot(ad_ref[...], zT, preferred_element_type=_F32)
    es_ref[...] = es
    ed_ref[...] = ed
    g_ref[...] = jnp.full((1, _L), jnp.max(es), _F32)


def _tc_entry(xT, W0, a_s, a_d):
    D, N = xT.shape
    H = W0.shape[1]
    return pl.pallas_call(
        _entry_body,
        out_shape=(
            jax.ShapeDtypeStruct((H, N), _F32),
            jax.ShapeDtypeStruct((1, N), _F32),
            jax.ShapeDtypeStruct((1, N), _F32),
            jax.ShapeDtypeStruct((1, _L), _F32),
        ),
    )(xT, W0, a_s.reshape(1, -1), a_d.reshape(1, -1))


def _mid_body(transpose_w, layer4, emit_h, refs):
    if emit_h:
        (num_ref, sp_ref, w_ref, as_ref, ad_ref,
         zT_ref, es_ref, ed_ref, g_ref, hT_ref) = refs
    else:
        (num_ref, sp_ref, w_ref, as_ref, ad_ref,
         zT_ref, es_ref, ed_ref, g_ref) = refs
    s = jnp.sum(sp_ref[...], axis=0, keepdims=True)
    num = jnp.sum(num_ref[...], axis=0)
    hT = _elu(num / (s + 1e-9))
    if emit_h:
        hT_ref[...] = hT
    if layer4:
        zT_ref[...] = hT
        wa_s = jnp.dot(as_ref[...], w_ref[...], preferred_element_type=_F32)
        wa_d = jnp.dot(ad_ref[...], w_ref[...], preferred_element_type=_F32)
        es = jnp.dot(wa_s, hT, preferred_element_type=_F32)
        ed = jnp.dot(wa_d, hT, preferred_element_type=_F32)
    else:
        cdim = 1 if transpose_w else 0
        zT = lax.dot_general(w_ref[...], hT, (((cdim,), (0,)), ((), ())),
                             preferred_element_type=_F32)
        zT_ref[...] = zT
        es = jnp.dot(as_ref[...], zT, preferred_element_type=_F32)
        ed = jnp.dot(ad_ref[...], zT, preferred_element_type=_F32)
    es_ref[...] = es
    ed_ref[...] = ed
    g_ref[...] = jnp.full((1, _L), jnp.max(es), _F32)


def _tc_mid(num, sp, W, a_s, a_d, *, transpose_w, layer4, emit_h):
    N = sp.shape[1]
    H = 16
    out_shape = [
        jax.ShapeDtypeStruct((H, N), _F32),
        jax.ShapeDtypeStruct((1, N), _F32),
        jax.ShapeDtypeStruct((1, N), _F32),
        jax.ShapeDtypeStruct((1, _L), _F32),
    ]
    if emit_h:
        out_shape.append(jax.ShapeDtypeStruct((16, N), _F32))

    def body(*refs):
        _mid_body(transpose_w, layer4, emit_h, refs)

    return pl.pallas_call(body, out_shape=tuple(out_shape))(
        num, sp, W, a_s.reshape(1, -1), a_d.reshape(1, -1))


def _final_body(num_ref, sp_ref, w_ref, out_ref):
    s = jnp.sum(sp_ref[...], axis=0, keepdims=True)
    agg = jnp.sum(num_ref[...], axis=0) / (s + 1e-9)
    reconT = lax.dot_general(w_ref[...], agg, (((1,), (0,)), ((), ())),
                             preferred_element_type=_F32)
    out_ref[...] = _elu(reconT)


def _tc_final(num, sp, W0):
    N = sp.shape[1]
    D = W0.shape[0]
    return pl.pallas_call(
        _final_body,
        out_shape=jax.ShapeDtypeStruct((D, N), _F32),
    )(num, sp, W0)


def _t_body(x_ref, o_ref):
    o_ref[...] = x_ref[...].T


def _transpose_big(x):
    A, B = x.shape
    return pl.pallas_call(
        _t_body,
        out_shape=jax.ShapeDtypeStruct((B, A), x.dtype),
    )(x)


def _sc_edge_body(N, E, src_hbm, dst_hbm, es_hbm, ed_hbm, g_hbm, vT_hbm,
                  num_hbm, s_hbm,
                  es_v, ed_v, g_v, z_v, acc_v, s_v, src_v, dst_v, ex_v, ex_sh):
    e_half = E // _NC
    ept1 = e_half // _NS
    eq_sz = e_half // _FPT
    ngrp = _CH // _L

    c = lax.axis_index("c")
    t = lax.axis_index("s")
    e0 = c * e_half

    pltpu.sync_copy(es_hbm, es_v)
    pltpu.sync_copy(ed_hbm, ed_v)
    pltpu.sync_copy(g_hbm, g_v)

    def zero_s(i, _):
        for u in range(5):
            s_v[pl.ds((i * 5 + u) * _L, _L)] = jnp.zeros((_L,), _F32)
        return 0

    lax.fori_loop(0, N // _L // 5, zero_s, 0)
    g = g_v[...]

    p1base = e0 + t * ept1

    def p1_chunk(k, _):
        b = p1base + k * _CH
        pltpu.sync_copy(src_hbm.at[pl.ds(b, _CH)], src_v)
        pltpu.sync_copy(dst_hbm.at[pl.ds(b, _CH)], dst_v)

        @plsc.parallel_loop(0, ngrp, 1, unroll=5)
        def _p1grp(gi):
            o = gi * _L
            sv = src_v[pl.ds(o, _L)]
            dv = dst_v[pl.ds(o, _L)]
            a = plsc.load_gather(es_v, [sv])
            bd = plsc.load_gather(ed_v, [dv])
            u = a + bd
            e = jnp.maximum(u, 0.2 * u)
            tq = bd + g
            q = jnp.maximum(tq, 0.2 * tq)
            ex = jnp.exp(e - q)
            ex_v[pl.ds(o, _L)] = ex
            plsc.addupdate_scatter(s_v, [dv], ex)
        pltpu.sync_copy(ex_v, ex_sh.at[pl.ds(t * ept1 + k * _CH, _CH)])
        return 0

    lax.fori_loop(0, ept1 // _CH, p1_chunk, 0)
    pltpu.sync_copy(s_v, s_hbm.at[c * _NS + t])
    plsc.subcore_barrier()

    fq = t % (_NS // _FPT)
    eq = t // (_NS // _FPT)
    fb = fq * _FPT
    pltpu.sync_copy(vT_hbm.at[pl.ds(fb, _FPT)], z_v)

    def zero_acc(i, _):
        for u in range(5):
            for j in range(_FPT):
                acc_v[j, pl.ds((i * 5 + u) * _L, _L)] = jnp.zeros((_L,), _F32)
        return 0

    lax.fori_loop(0, N // _L // 5, zero_acc, 0)

    p2local = eq * eq_sz

    def p2_chunk(k, _):
        bl = p2local + k * _CH
        b = e0 + bl
        pltpu.sync_copy(src_hbm.at[pl.ds(b, _CH)], src_v)
        pltpu.sync_copy(dst_hbm.at[pl.ds(b, _CH)], dst_v)
        pltpu.sync_copy(ex_sh.at[pl.ds(bl, _CH)], ex_v)

        @plsc.parallel_loop(0, ngrp, 1, unroll=5)
        def _p2grp(gi):
            o = gi * _L
            sv = src_v[pl.ds(o, _L)]
            dv = dst_v[pl.ds(o, _L)]
            exv = ex_v[pl.ds(o, _L)]
            for j in range(_FPT):
                rj = jnp.full((_L,), j, jnp.int32)
                zg = plsc.load_gather(z_v, [rj, sv])
                plsc.addupdate_scatter(acc_v, [rj, dv], zg * exv)
        return 0

    lax.fori_loop(0, eq_sz // _CH, p2_chunk, 0)
    pltpu.sync_copy(acc_v, num_hbm.at[c * _FPT + eq, pl.ds(fb, _FPT)])


@functools.partial(jax.jit, static_argnames=("N", "E"))
def _sc_edge(src, dst, es, ed, g, vT, *, N, E):
    mesh = plsc.VectorSubcoreMesh(core_axis_name="c", subcore_axis_name="s",
                                  num_cores=_NC, num_subcores=_NS)
    body = functools.partial(_sc_edge_body, N, E)
    kern = pl.kernel(
        body,
        out_type=(
            jax.ShapeDtypeStruct((_NC * _FPT, 16, N), _F32),
            jax.ShapeDtypeStruct((_NC * _NS, N), _F32),
        ),
        mesh=mesh,
        compiler_params=pltpu.CompilerParams(needs_layout_passes=False),
        scratch_types=[
            pltpu.VMEM((N,), _F32),
            pltpu.VMEM((N,), _F32),
            pltpu.VMEM((_L,), _F32),
            pltpu.VMEM((_FPT, N), _F32),
            pltpu.VMEM((_FPT, N), _F32),
            pltpu.VMEM((N,), _F32),
            pltpu.VMEM((_CH,), jnp.int32),
            pltpu.VMEM((_CH,), jnp.int32),
            pltpu.VMEM((_CH,), _F32),
            pltpu.VMEM_SHARED((E // _NC,), _F32),
        ],
    )
    return kern(src, dst, es, ed, g, vT)


def kernel(x, edge_index, W0, W1,
           a_enc0_src, a_enc0_dst, a_enc1_src, a_enc1_dst,
           a_dec0_src, a_dec0_dst, a_dec1_src, a_dec1_dst):
    N, D = x.shape
    E = edge_index.shape[1]
    src = edge_index[0]
    dst = edge_index[1]

    xT = _transpose_big(x)

    zT, es, ed, g = _tc_entry(xT, W0, a_enc0_src, a_enc0_dst)
    num, sp = _sc_edge(src, dst, es.reshape(-1), ed.reshape(-1),
                       g.reshape(-1), zT, N=N, E=E)

    zT, es, ed, g = _tc_mid(num, sp, W1, a_enc1_src, a_enc1_dst,
                            transpose_w=False, layer4=False, emit_h=False)
    num, sp = _sc_edge(src, dst, es.reshape(-1), ed.reshape(-1),
                       g.reshape(-1), zT, N=N, E=E)

    zT, es, ed, g, hiddenT = _tc_mid(num, sp, W1, a_dec0_src, a_dec0_dst,
                                     transpose_w=True, layer4=False,
                                     emit_h=True)
    num, sp = _sc_edge(src, dst, es.reshape(-1), ed.reshape(-1),
                       g.reshape(-1), zT, N=N, E=E)

    rT, es, ed, g = _tc_mid(num, sp, W0, a_dec1_src, a_dec1_dst,
                            transpose_w=False, layer4=True, emit_h=False)
    num, sp = _sc_edge(src, dst, es.reshape(-1), ed.reshape(-1),
                       g.reshape(-1), rT, N=N, E=E)
    reconT = _tc_final(num, sp, W0)

    hidden = _transpose_big(hiddenT)
    recon = _transpose_big(reconT)
    return (hidden, recon)


# speedup vs baseline: 116.0323x; 1.5080x over previous
import functools

import jax
import jax.numpy as jnp
from jax import lax
from jax.experimental import pallas as pl
from jax.experimental.pallas import tpu as pltpu
from jax.experimental.pallas import tpu_sc as plsc

_NC = 2
_NS = 16
_L = 16
_FPT = 4
_CH = 2000

_F32 = jnp.float32


def _elu(x):
    return jnp.where(x > 0, x, jnp.exp(jnp.minimum(x, 0.0)) - 1.0)


def _entry_body(xT_ref, w_ref, as_ref, ad_ref, zT_ref, es_ref, ed_ref, g_ref):
    xT = xT_ref[...]
    w = w_ref[...]
    zT = lax.dot_general(w, xT, (((0,), (0,)), ((), ())),
                         preferred_element_type=_F32)
    zT_ref[...] = zT
    es = jnp.dot(as_ref[...], zT, preferred_element_type=_F32)
    ed = jnp.dot(ad_ref[...], zT, preferred_element_type=_F32)
    es_ref[...] = es
    ed_ref[...] = ed
    g_ref[...] = jnp.full((1, _L), jnp.max(es), _F32)


def _tc_entry(xT, W0, a_s, a_d):
    D, N = xT.shape
    H = W0.shape[1]
    return pl.pallas_call(
        _entry_body,
        out_shape=(
            jax.ShapeDtypeStruct((H, N), _F32),
            jax.ShapeDtypeStruct((1, N), _F32),
            jax.ShapeDtypeStruct((1, N), _F32),
            jax.ShapeDtypeStruct((1, _L), _F32),
        ),
    )(xT, W0, a_s.reshape(1, -1), a_d.reshape(1, -1))


def _mid_body(transpose_w, layer4, emit_h, refs):
    if emit_h:
        (num_ref, sp_ref, w_ref, as_ref, ad_ref,
         zT_ref, es_ref, ed_ref, g_ref, hT_ref) = refs
    else:
        (num_ref, sp_ref, w_ref, as_ref, ad_ref,
         zT_ref, es_ref, ed_ref, g_ref) = refs
    s = 0.25 * jnp.sum(sp_ref[...], axis=0, keepdims=True)
    num = jnp.sum(num_ref[...], axis=0)
    hT = _elu(num / (s + 1e-9))
    if emit_h:
        hT_ref[...] = hT
    if layer4:
        zT_ref[...] = hT
        wa_s = jnp.dot(as_ref[...], w_ref[...], preferred_element_type=_F32)
        wa_d = jnp.dot(ad_ref[...], w_ref[...], preferred_element_type=_F32)
        es = jnp.dot(wa_s, hT, preferred_element_type=_F32)
        ed = jnp.dot(wa_d, hT, preferred_element_type=_F32)
    else:
        cdim = 1 if transpose_w else 0
        zT = lax.dot_general(w_ref[...], hT, (((cdim,), (0,)), ((), ())),
                             preferred_element_type=_F32)
        zT_ref[...] = zT
        es = jnp.dot(as_ref[...], zT, preferred_element_type=_F32)
        ed = jnp.dot(ad_ref[...], zT, preferred_element_type=_F32)
    es_ref[...] = es
    ed_ref[...] = ed
    g_ref[...] = jnp.full((1, _L), jnp.max(es), _F32)


def _tc_mid(num, sp, W, a_s, a_d, *, transpose_w, layer4, emit_h):
    N = sp.shape[1]
    H = 16
    out_shape = [
        jax.ShapeDtypeStruct((H, N), _F32),
        jax.ShapeDtypeStruct((1, N), _F32),
        jax.ShapeDtypeStruct((1, N), _F32),
        jax.ShapeDtypeStruct((1, _L), _F32),
    ]
    if emit_h:
        out_shape.append(jax.ShapeDtypeStruct((16, N), _F32))

    def body(*refs):
        _mid_body(transpose_w, layer4, emit_h, refs)

    return pl.pallas_call(body, out_shape=tuple(out_shape))(
        num, sp, W, a_s.reshape(1, -1), a_d.reshape(1, -1))


def _final_body(num_ref, sp_ref, w_ref, out_ref):
    s = 0.25 * jnp.sum(sp_ref[...], axis=0, keepdims=True)
    agg = jnp.sum(num_ref[...], axis=0) / (s + 1e-9)
    reconT = lax.dot_general(w_ref[...], agg, (((1,), (0,)), ((), ())),
                             preferred_element_type=_F32)
    out_ref[...] = _elu(reconT)


def _tc_final(num, sp, W0):
    N = sp.shape[1]
    D = W0.shape[0]
    return pl.pallas_call(
        _final_body,
        out_shape=jax.ShapeDtypeStruct((D, N), _F32),
    )(num, sp, W0)


def _t_body(x_ref, o_ref):
    o_ref[...] = x_ref[...].T


def _transpose_big(x):
    A, B = x.shape
    return pl.pallas_call(
        _t_body,
        out_shape=jax.ShapeDtypeStruct((B, A), x.dtype),
    )(x)


def _sc_edge_body(N, E, src_hbm, dst_hbm, es_hbm, ed_hbm, g_hbm, vT_hbm,
                  num_hbm, s_hbm,
                  es_v, ed_v, g_v, z_v, acc_v, s_v,
                  src_a, dst_a, src_b, dst_b, sem_a, sem_b, sem_w):
    e_half = E // _NC
    eq_sz = e_half // _FPT
    ngrp = _CH // _L
    nch = eq_sz // _CH

    c = lax.axis_index("c")
    t = lax.axis_index("s")
    e0 = c * e_half
    fq = t % (_NS // _FPT)
    eq = t // (_NS // _FPT)
    fb = fq * _FPT

    d_es = pltpu.async_copy(es_hbm, es_v, sem_w)
    d_ed = pltpu.async_copy(ed_hbm, ed_v, sem_w)
    d_g = pltpu.async_copy(g_hbm, g_v, sem_w)
    d_z = pltpu.async_copy(vT_hbm.at[pl.ds(fb, _FPT)], z_v, sem_w)

    @plsc.parallel_loop(0, N // _L, 1, unroll=8)
    def _zs(i):
        s_v[pl.ds(i * _L, _L)] = jnp.zeros((_L,), _F32)

    @plsc.parallel_loop(0, N // _L, 1, unroll=8)
    def _za(i):
        for j in range(_FPT):
            acc_v[j, pl.ds(i * _L, _L)] = jnp.zeros((_L,), _F32)

    d_es.wait()
    d_ed.wait()
    d_g.wait()
    d_z.wait()
    g = g_v[...]

    bufs = ((src_a, dst_a, sem_a), (src_b, dst_b, sem_b))
    p0 = e0 + eq * eq_sz

    def start(k):
        sv, dv, sm = bufs[k % 2]
        b = p0 + k * _CH
        return (pltpu.async_copy(src_hbm.at[pl.ds(b, _CH)], sv, sm),
                pltpu.async_copy(dst_hbm.at[pl.ds(b, _CH)], dv, sm))

    pend = {0: start(0)}
    for k in range(nch):
        if k + 1 < nch:
            pend[k + 1] = start(k + 1)
        for d in pend.pop(k):
            d.wait()
        sv, dv, _ = bufs[k % 2]

        @plsc.parallel_loop(0, ngrp, 1, unroll=5)
        def _grp(gi, _sv=sv, _dv=dv):
            o = gi * _L
            svv = _sv[pl.ds(o, _L)]
            dvv = _dv[pl.ds(o, _L)]
            a = plsc.load_gather(es_v, [svv])
            bd = plsc.load_gather(ed_v, [dvv])
            u = a + bd
            e = jnp.maximum(u, 0.2 * u)
            tq = bd + g
            q = jnp.maximum(tq, 0.2 * tq)
            ex = jnp.exp(e - q)
            plsc.addupdate_scatter(s_v, [dvv], ex)
            for j in range(_FPT):
                rj = jnp.full((_L,), j, jnp.int32)
                zg = plsc.load_gather(z_v, [rj, svv])
                plsc.addupdate_scatter(acc_v, [rj, dvv], zg * ex)

    pltpu.sync_copy(s_v, s_hbm.at[c * _NS + t])
    pltpu.sync_copy(acc_v, num_hbm.at[c * _FPT + eq, pl.ds(fb, _FPT)])


@functools.partial(jax.jit, static_argnames=("N", "E"))
def _sc_edge(src, dst, es, ed, g, vT, *, N, E):
    mesh = plsc.VectorSubcoreMesh(core_axis_name="c", subcore_axis_name="s",
                                  num_cores=_NC, num_subcores=_NS)
    body = functools.partial(_sc_edge_body, N, E)
    kern = pl.kernel(
        body,
        out_type=(
            jax.ShapeDtypeStruct((_NC * _FPT, 16, N), _F32),
            jax.ShapeDtypeStruct((_NC * _NS, N), _F32),
        ),
        mesh=mesh,
        compiler_params=pltpu.CompilerParams(needs_layout_passes=False),
        scratch_types=[
            pltpu.VMEM((N,), _F32),
            pltpu.VMEM((N,), _F32),
            pltpu.VMEM((_L,), _F32),
            pltpu.VMEM((_FPT, N), _F32),
            pltpu.VMEM((_FPT, N), _F32),
            pltpu.VMEM((N,), _F32),
            pltpu.VMEM((_CH,), jnp.int32),
            pltpu.VMEM((_CH,), jnp.int32),
            pltpu.VMEM((_CH,), jnp.int32),
            pltpu.VMEM((_CH,), jnp.int32),
            pltpu.SemaphoreType.DMA,
            pltpu.SemaphoreType.DMA,
            pltpu.SemaphoreType.DMA,
        ],
    )
    return kern(src, dst, es, ed, g, vT)


def kernel(x, edge_index, W0, W1,
           a_enc0_src, a_enc0_dst, a_enc1_src, a_enc1_dst,
           a_dec0_src, a_dec0_dst, a_dec1_src, a_dec1_dst):
    N, D = x.shape
    E = edge_index.shape[1]
    src = edge_index[0]
    dst = edge_index[1]

    xT = _transpose_big(x)

    zT, es, ed, g = _tc_entry(xT, W0, a_enc0_src, a_enc0_dst)
    num, sp = _sc_edge(src, dst, es.reshape(-1), ed.reshape(-1),
                       g.reshape(-1), zT, N=N, E=E)

    zT, es, ed, g = _tc_mid(num, sp, W1, a_enc1_src, a_enc1_dst,
                            transpose_w=False, layer4=False, emit_h=False)
    num, sp = _sc_edge(src, dst, es.reshape(-1), ed.reshape(-1),
                       g.reshape(-1), zT, N=N, E=E)

    zT, es, ed, g, hiddenT = _tc_mid(num, sp, W1, a_dec0_src, a_dec0_dst,
                                     transpose_w=True, layer4=False,
                                     emit_h=True)
    num, sp = _sc_edge(src, dst, es.reshape(-1), ed.reshape(-1),
                       g.reshape(-1), zT, N=N, E=E)

    rT, es, ed, g = _tc_mid(num, sp, W0, a_dec1_src, a_dec1_dst,
                            transpose_w=False, layer4=True, emit_h=False)
    num, sp = _sc_edge(src, dst, es.reshape(-1), ed.reshape(-1),
                       g.reshape(-1), rT, N=N, E=E)
    reconT = _tc_final(num, sp, W0)

    hidden = _transpose_big(hiddenT)
    recon = _transpose_big(reconT)
    return (hidden, recon)


# speedup vs baseline: 121.2387x; 1.0449x over previous
import functools

import jax
import jax.numpy as jnp
from jax import lax
from jax.experimental import pallas as pl
from jax.experimental.pallas import tpu as pltpu
from jax.experimental.pallas import tpu_sc as plsc

_NC = 2
_NS = 16
_L = 16
_FPT = 4
_CH = 2000

_F32 = jnp.float32


def _elu(x):
    return jnp.where(x > 0, x, jnp.exp(jnp.minimum(x, 0.0)) - 1.0)


def _entry_body(x_ref, w_ref, as_ref, ad_ref, zT_ref, es_ref, ed_ref, g_ref):
    x = x_ref[...]
    w = w_ref[...]
    zT = lax.dot_general(w, x, (((0,), (1,)), ((), ())),
                         preferred_element_type=_F32)
    zT_ref[...] = zT
    es = jnp.dot(as_ref[...], zT, preferred_element_type=_F32)
    ed = jnp.dot(ad_ref[...], zT, preferred_element_type=_F32)
    es_ref[...] = es
    ed_ref[...] = ed
    g_ref[...] = jnp.full((1, _L), jnp.max(es), _F32)


def _tc_entry(x, W0, a_s, a_d):
    N, D = x.shape
    H = W0.shape[1]
    return pl.pallas_call(
        _entry_body,
        out_shape=(
            jax.ShapeDtypeStruct((H, N), _F32),
            jax.ShapeDtypeStruct((1, N), _F32),
            jax.ShapeDtypeStruct((1, N), _F32),
            jax.ShapeDtypeStruct((1, _L), _F32),
        ),
    )(x, W0, a_s.reshape(1, -1), a_d.reshape(1, -1))


def _mid_body(transpose_w, layer4, emit_h, refs):
    if emit_h:
        (num_ref, sp_ref, w_ref, as_ref, ad_ref,
         zT_ref, es_ref, ed_ref, g_ref, hT_ref) = refs
    else:
        (num_ref, sp_ref, w_ref, as_ref, ad_ref,
         zT_ref, es_ref, ed_ref, g_ref) = refs
    s = 0.25 * jnp.sum(sp_ref[...], axis=0, keepdims=True)
    num = jnp.sum(num_ref[...], axis=0)
    hT = _elu(num / (s + 1e-9))
    if emit_h:
        hT_ref[...] = hT.T
    if layer4:
        zT_ref[...] = hT
        wa_s = jnp.dot(as_ref[...], w_ref[...], preferred_element_type=_F32)
        wa_d = jnp.dot(ad_ref[...], w_ref[...], preferred_element_type=_F32)
        es = jnp.dot(wa_s, hT, preferred_element_type=_F32)
        ed = jnp.dot(wa_d, hT, preferred_element_type=_F32)
    else:
        cdim = 1 if transpose_w else 0
        zT = lax.dot_general(w_ref[...], hT, (((cdim,), (0,)), ((), ())),
                             preferred_element_type=_F32)
        zT_ref[...] = zT
        es = jnp.dot(as_ref[...], zT, preferred_element_type=_F32)
        ed = jnp.dot(ad_ref[...], zT, preferred_element_type=_F32)
    es_ref[...] = es
    ed_ref[...] = ed
    g_ref[...] = jnp.full((1, _L), jnp.max(es), _F32)


def _tc_mid(num, sp, W, a_s, a_d, *, transpose_w, layer4, emit_h):
    N = sp.shape[1]
    H = 16
    out_shape = [
        jax.ShapeDtypeStruct((H, N), _F32),
        jax.ShapeDtypeStruct((1, N), _F32),
        jax.ShapeDtypeStruct((1, N), _F32),
        jax.ShapeDtypeStruct((1, _L), _F32),
    ]
    if emit_h:
        out_shape.append(jax.ShapeDtypeStruct((N, 16), _F32))

    def body(*refs):
        _mid_body(transpose_w, layer4, emit_h, refs)

    return pl.pallas_call(body, out_shape=tuple(out_shape))(
        num, sp, W, a_s.reshape(1, -1), a_d.reshape(1, -1))


def _final_body(num_ref, sp_ref, w_ref, out_ref):
    s = 0.25 * jnp.sum(sp_ref[...], axis=0, keepdims=True)
    agg = jnp.sum(num_ref[...], axis=0) / (s + 1e-9)
    recon = lax.dot_general(agg, w_ref[...], (((0,), (1,)), ((), ())),
                            preferred_element_type=_F32)
    out_ref[...] = _elu(recon)


def _tc_final(num, sp, W0):
    N = sp.shape[1]
    D = W0.shape[0]
    return pl.pallas_call(
        _final_body,
        out_shape=jax.ShapeDtypeStruct((N, D), _F32),
    )(num, sp, W0)


def _sc_edge_body(N, E, src_hbm, dst_hbm, es_hbm, ed_hbm, g_hbm, vT_hbm,
                  num_hbm, s_hbm,
                  es_v, ed_v, g_v, z_v, acc_v, s_v,
                  src_a, dst_a, src_b, dst_b, sem_a, sem_b, sem_w):
    e_half = E // _NC
    eq_sz = e_half // _FPT
    ngrp = _CH // _L
    nch = eq_sz // _CH

    c = lax.axis_index("c")
    t = lax.axis_index("s")
    e0 = c * e_half
    fq = t % (_NS // _FPT)
    eq = t // (_NS // _FPT)
    fb = fq * _FPT

    d_es = pltpu.async_copy(es_hbm, es_v, sem_w)
    d_ed = pltpu.async_copy(ed_hbm, ed_v, sem_w)
    d_g = pltpu.async_copy(g_hbm, g_v, sem_w)
    d_z = pltpu.async_copy(vT_hbm.at[pl.ds(fb, _FPT)], z_v, sem_w)

    @plsc.parallel_loop(0, N // _L, 1, unroll=8)
    def _zs(i):
        s_v[pl.ds(i * _L, _L)] = jnp.zeros((_L,), _F32)

    @plsc.parallel_loop(0, N // _L, 1, unroll=8)
    def _za(i):
        for j in range(_FPT):
            acc_v[j, pl.ds(i * _L, _L)] = jnp.zeros((_L,), _F32)

    d_es.wait()
    d_ed.wait()
    d_g.wait()
    d_z.wait()
    g = g_v[...]

    bufs = ((src_a, dst_a, sem_a), (src_b, dst_b, sem_b))
    p0 = e0 + eq * eq_sz

    def start(k):
        sv, dv, sm = bufs[k % 2]
        b = p0 + k * _CH
        return (pltpu.async_copy(src_hbm.at[pl.ds(b, _CH)], sv, sm),
                pltpu.async_copy(dst_hbm.at[pl.ds(b, _CH)], dv, sm))

    pend = {0: start(0)}
    for k in range(nch):
        if k + 1 < nch:
            pend[k + 1] = start(k + 1)
        for d in pend.pop(k):
            d.wait()
        sv, dv, _ = bufs[k % 2]

        @plsc.parallel_loop(0, ngrp, 1, unroll=5)
        def _grp(gi, _sv=sv, _dv=dv):
            o = gi * _L
            svv = _sv[pl.ds(o, _L)]
            dvv = _dv[pl.ds(o, _L)]
            a = plsc.load_gather(es_v, [svv])
            bd = plsc.load_gather(ed_v, [dvv])
            u = a + bd
            e = jnp.maximum(u, 0.2 * u)
            tq = bd + g
            q = jnp.maximum(tq, 0.2 * tq)
            ex = jnp.exp(e - q)
            plsc.addupdate_scatter(s_v, [dvv], ex)
            for j in range(_FPT):
                rj = jnp.full((_L,), j, jnp.int32)
                zg = plsc.load_gather(z_v, [rj, svv])
                plsc.addupdate_scatter(acc_v, [rj, dvv], zg * ex)

    pltpu.sync_copy(s_v, s_hbm.at[c * _NS + t])
    pltpu.sync_copy(acc_v, num_hbm.at[c * _FPT + eq, pl.ds(fb, _FPT)])


@functools.partial(jax.jit, static_argnames=("N", "E"))
def _sc_edge(src, dst, es, ed, g, vT, *, N, E):
    mesh = plsc.VectorSubcoreMesh(core_axis_name="c", subcore_axis_name="s",
                                  num_cores=_NC, num_subcores=_NS)
    body = functools.partial(_sc_edge_body, N, E)
    kern = pl.kernel(
        body,
        out_type=(
            jax.ShapeDtypeStruct((_NC * _FPT, 16, N), _F32),
            jax.ShapeDtypeStruct((_NC * _NS, N), _F32),
        ),
        mesh=mesh,
        compiler_params=pltpu.CompilerParams(needs_layout_passes=False),
        scratch_types=[
            pltpu.VMEM((N,), _F32),
            pltpu.VMEM((N,), _F32),
            pltpu.VMEM((_L,), _F32),
            pltpu.VMEM((_FPT, N), _F32),
            pltpu.VMEM((_FPT, N), _F32),
            pltpu.VMEM((N,), _F32),
            pltpu.VMEM((_CH,), jnp.int32),
            pltpu.VMEM((_CH,), jnp.int32),
            pltpu.VMEM((_CH,), jnp.int32),
            pltpu.VMEM((_CH,), jnp.int32),
            pltpu.SemaphoreType.DMA,
            pltpu.SemaphoreType.DMA,
            pltpu.SemaphoreType.DMA,
        ],
    )
    return kern(src, dst, es, ed, g, vT)


def kernel(x, edge_index, W0, W1,
           a_enc0_src, a_enc0_dst, a_enc1_src, a_enc1_dst,
           a_dec0_src, a_dec0_dst, a_dec1_src, a_dec1_dst):
    N, D = x.shape
    E = edge_index.shape[1]
    src = edge_index[0]
    dst = edge_index[1]

    zT, es, ed, g = _tc_entry(x, W0, a_enc0_src, a_enc0_dst)
    num, sp = _sc_edge(src, dst, es.reshape(-1), ed.reshape(-1),
                       g.reshape(-1), zT, N=N, E=E)

    zT, es, ed, g = _tc_mid(num, sp, W1, a_enc1_src, a_enc1_dst,
                            transpose_w=False, layer4=False, emit_h=False)
    num, sp = _sc_edge(src, dst, es.reshape(-1), ed.reshape(-1),
                       g.reshape(-1), zT, N=N, E=E)

    zT, es, ed, g, hidden = _tc_mid(num, sp, W1, a_dec0_src, a_dec0_dst,
                                    transpose_w=True, layer4=False,
                                    emit_h=True)
    num, sp = _sc_edge(src, dst, es.reshape(-1), ed.reshape(-1),
                       g.reshape(-1), zT, N=N, E=E)

    rT, es, ed, g = _tc_mid(num, sp, W0, a_dec1_src, a_dec1_dst,
                            transpose_w=False, layer4=True, emit_h=False)
    num, sp = _sc_edge(src, dst, es.reshape(-1), ed.reshape(-1),
                       g.reshape(-1), rT, N=N, E=E)
    recon = _tc_final(num, sp, W0)

    return (hidden, recon)


# speedup vs baseline: 125.7521x; 1.0372x over previous
import functools

import jax
import jax.numpy as jnp
from jax import lax
from jax.experimental import pallas as pl
from jax.experimental.pallas import tpu as pltpu
from jax.experimental.pallas import tpu_sc as plsc

_NC = 2
_NS = 16
_L = 16
_FPT = 4
_CH = 4000

_F32 = jnp.float32


def _elu(x):
    return jnp.where(x > 0, x, jnp.exp(jnp.minimum(x, 0.0)) - 1.0)


def _entry_body(x_ref, w_ref, as_ref, ad_ref, zT_ref, es_ref, ed_ref, g_ref):
    x = x_ref[...]
    w = w_ref[...]
    zT = lax.dot_general(w, x, (((0,), (1,)), ((), ())),
                         preferred_element_type=_F32)
    zT_ref[...] = zT
    es = jnp.dot(as_ref[...], zT, preferred_element_type=_F32)
    ed = jnp.dot(ad_ref[...], zT, preferred_element_type=_F32)
    es_ref[...] = es
    ed_ref[...] = ed
    g_ref[...] = jnp.full((1, _L), jnp.max(es), _F32)


def _tc_entry(x, W0, a_s, a_d):
    N, D = x.shape
    H = W0.shape[1]
    return pl.pallas_call(
        _entry_body,
        out_shape=(
            jax.ShapeDtypeStruct((H, N), _F32),
            jax.ShapeDtypeStruct((1, N), _F32),
            jax.ShapeDtypeStruct((1, N), _F32),
            jax.ShapeDtypeStruct((1, _L), _F32),
        ),
    )(x, W0, a_s.reshape(1, -1), a_d.reshape(1, -1))


def _mid_body(transpose_w, layer4, emit_h, refs):
    if emit_h:
        (num_ref, sp_ref, w_ref, as_ref, ad_ref,
         zT_ref, es_ref, ed_ref, g_ref, hT_ref) = refs
    else:
        (num_ref, sp_ref, w_ref, as_ref, ad_ref,
         zT_ref, es_ref, ed_ref, g_ref) = refs
    s = 0.25 * jnp.sum(sp_ref[...], axis=0, keepdims=True)
    num = jnp.sum(num_ref[...], axis=0)
    hT = _elu(num / (s + 1e-9))
    if emit_h:
        hT_ref[...] = hT.T
    if layer4:
        zT_ref[...] = hT
        wa_s = jnp.dot(as_ref[...], w_ref[...], preferred_element_type=_F32)
        wa_d = jnp.dot(ad_ref[...], w_ref[...], preferred_element_type=_F32)
        es = jnp.dot(wa_s, hT, preferred_element_type=_F32)
        ed = jnp.dot(wa_d, hT, preferred_element_type=_F32)
    else:
        cdim = 1 if transpose_w else 0
        zT = lax.dot_general(w_ref[...], hT, (((cdim,), (0,)), ((), ())),
                             preferred_element_type=_F32)
        zT_ref[...] = zT
        es = jnp.dot(as_ref[...], zT, preferred_element_type=_F32)
        ed = jnp.dot(ad_ref[...], zT, preferred_element_type=_F32)
    es_ref[...] = es
    ed_ref[...] = ed
    g_ref[...] = jnp.full((1, _L), jnp.max(es), _F32)


def _tc_mid(num, sp, W, a_s, a_d, *, transpose_w, layer4, emit_h):
    N = sp.shape[1]
    H = 16
    out_shape = [
        jax.ShapeDtypeStruct((H, N), _F32),
        jax.ShapeDtypeStruct((1, N), _F32),
        jax.ShapeDtypeStruct((1, N), _F32),
        jax.ShapeDtypeStruct((1, _L), _F32),
    ]
    if emit_h:
        out_shape.append(jax.ShapeDtypeStruct((N, 16), _F32))

    def body(*refs):
        _mid_body(transpose_w, layer4, emit_h, refs)

    return pl.pallas_call(body, out_shape=tuple(out_shape))(
        num, sp, W, a_s.reshape(1, -1), a_d.reshape(1, -1))


def _final_body(num_ref, sp_ref, w_ref, out_ref):
    s = 0.25 * jnp.sum(sp_ref[...], axis=0, keepdims=True)
    agg = jnp.sum(num_ref[...], axis=0) / (s + 1e-9)
    recon = lax.dot_general(agg, w_ref[...], (((0,), (1,)), ((), ())),
                            preferred_element_type=_F32)
    out_ref[...] = _elu(recon)


def _tc_final(num, sp, W0):
    N = sp.shape[1]
    D = W0.shape[0]
    return pl.pallas_call(
        _final_body,
        out_shape=jax.ShapeDtypeStruct((N, D), _F32),
    )(num, sp, W0)


def _sc_edge_body(N, E, src_hbm, dst_hbm, es_hbm, ed_hbm, g_hbm, vT_hbm,
                  num_hbm, s_hbm,
                  es_v, ed_v, g_v, z_v, acc_v, s_v,
                  src_a, dst_a, src_b, dst_b, sem_a, sem_b, sem_w):
    e_half = E // _NC
    eq_sz = e_half // _FPT
    ngrp = _CH // _L
    nch = eq_sz // _CH

    c = lax.axis_index("c")
    t = lax.axis_index("s")
    e0 = c * e_half
    fq = t % (_NS // _FPT)
    eq = t // (_NS // _FPT)
    fb = fq * _FPT

    d_es = pltpu.async_copy(es_hbm, es_v, sem_w)
    d_ed = pltpu.async_copy(ed_hbm, ed_v, sem_w)
    d_g = pltpu.async_copy(g_hbm, g_v, sem_w)
    d_z = pltpu.async_copy(vT_hbm.at[pl.ds(fb, _FPT)], z_v, sem_w)

    @plsc.parallel_loop(0, N // _L, 1, unroll=8)
    def _zs(i):
        s_v[pl.ds(i * _L, _L)] = jnp.zeros((_L,), _F32)

    @plsc.parallel_loop(0, N // _L, 1, unroll=8)
    def _za(i):
        for j in range(_FPT):
            acc_v[j, pl.ds(i * _L, _L)] = jnp.zeros((_L,), _F32)

    d_es.wait()
    d_ed.wait()
    d_g.wait()
    d_z.wait()
    g = g_v[...]

    bufs = ((src_a, dst_a, sem_a), (src_b, dst_b, sem_b))
    p0 = e0 + eq * eq_sz

    def start(k):
        sv, dv, sm = bufs[k % 2]
        b = p0 + k * _CH
        return (pltpu.async_copy(src_hbm.at[pl.ds(b, _CH)], sv, sm),
                pltpu.async_copy(dst_hbm.at[pl.ds(b, _CH)], dv, sm))

    pend = {0: start(0)}
    for k in range(nch):
        if k + 1 < nch:
            pend[k + 1] = start(k + 1)
        for d in pend.pop(k):
            d.wait()
        sv, dv, _ = bufs[k % 2]

        @plsc.parallel_loop(0, ngrp, 1, unroll=5)
        def _grp(gi, _sv=sv, _dv=dv):
            o = gi * _L
            svv = _sv[pl.ds(o, _L)]
            dvv = _dv[pl.ds(o, _L)]
            a = plsc.load_gather(es_v, [svv])
            bd = plsc.load_gather(ed_v, [dvv])
            u = a + bd
            e = jnp.maximum(u, 0.2 * u)
            tq = bd + g
            q = jnp.maximum(tq, 0.2 * tq)
            ex = jnp.exp(e - q)
            plsc.addupdate_scatter(s_v, [dvv], ex)
            for j in range(_FPT):
                rj = jnp.full((_L,), j, jnp.int32)
                zg = plsc.load_gather(z_v, [rj, svv])
                plsc.addupdate_scatter(acc_v, [rj, dvv], zg * ex)

    pltpu.sync_copy(s_v, s_hbm.at[c * _NS + t])
    pltpu.sync_copy(acc_v, num_hbm.at[c * _FPT + eq, pl.ds(fb, _FPT)])


@functools.partial(jax.jit, static_argnames=("N", "E"))
def _sc_edge(src, dst, es, ed, g, vT, *, N, E):
    mesh = plsc.VectorSubcoreMesh(core_axis_name="c", subcore_axis_name="s",
                                  num_cores=_NC, num_subcores=_NS)
    body = functools.partial(_sc_edge_body, N, E)
    kern = pl.kernel(
        body,
        out_type=(
            jax.ShapeDtypeStruct((_NC * _FPT, 16, N), _F32),
            jax.ShapeDtypeStruct((_NC * _NS, N), _F32),
        ),
        mesh=mesh,
        compiler_params=pltpu.CompilerParams(needs_layout_passes=False),
        scratch_types=[
            pltpu.VMEM((N,), _F32),
            pltpu.VMEM((N,), _F32),
            pltpu.VMEM((_L,), _F32),
            pltpu.VMEM((_FPT, N), _F32),
            pltpu.VMEM((_FPT, N), _F32),
            pltpu.VMEM((N,), _F32),
            pltpu.VMEM((_CH,), jnp.int32),
            pltpu.VMEM((_CH,), jnp.int32),
            pltpu.VMEM((_CH,), jnp.int32),
            pltpu.VMEM((_CH,), jnp.int32),
            pltpu.SemaphoreType.DMA,
            pltpu.SemaphoreType.DMA,
            pltpu.SemaphoreType.DMA,
        ],
    )
    return kern(src, dst, es, ed, g, vT)


def kernel(x, edge_index, W0, W1,
           a_enc0_src, a_enc0_dst, a_enc1_src, a_enc1_dst,
           a_dec0_src, a_dec0_dst, a_dec1_src, a_dec1_dst):
    N, D = x.shape
    E = edge_index.shape[1]
    src = edge_index[0]
    dst = edge_index[1]

    zT, es, ed, g = _tc_entry(x, W0, a_enc0_src, a_enc0_dst)
    num, sp = _sc_edge(src, dst, es.reshape(-1), ed.reshape(-1),
                       g.reshape(-1), zT, N=N, E=E)

    zT, es, ed, g = _tc_mid(num, sp, W1, a_enc1_src, a_enc1_dst,
                            transpose_w=False, layer4=False, emit_h=False)
    num, sp = _sc_edge(src, dst, es.reshape(-1), ed.reshape(-1),
                       g.reshape(-1), zT, N=N, E=E)

    zT, es, ed, g, hidden = _tc_mid(num, sp, W1, a_dec0_src, a_dec0_dst,
                                    transpose_w=True, layer4=False,
                                    emit_h=True)
    num, sp = _sc_edge(src, dst, es.reshape(-1), ed.reshape(-1),
                       g.reshape(-1), zT, N=N, E=E)

    rT, es, ed, g = _tc_mid(num, sp, W0, a_dec1_src, a_dec1_dst,
                            transpose_w=False, layer4=True, emit_h=False)
    num, sp = _sc_edge(src, dst, es.reshape(-1), ed.reshape(-1),
                       g.reshape(-1), rT, N=N, E=E)
    recon = _tc_final(num, sp, W0)

    return (hidden, recon)
